# baseline (speedup 1.0000x reference)
import functools

import jax
import jax.numpy as jnp
from jax import lax
from jax.experimental import pallas as pl
from jax.experimental.pallas import tpu as pltpu
from jax.experimental.pallas import tpu_sc as plsc

N = 1024
D = 32
M = 100000
K = 16
CHUNK = 256
NCHUNK = -(-M // CHUNK)
MPAD = CHUNK * NCHUNK
INT_MAX = jnp.iinfo(jnp.int32).max


def _topk_body(q_ref, t_ref, vals_ref, idx_ref):
    i = pl.program_id(0)

    st = lax.dot_general(
        t_ref[...], q_ref[...], (((1,), (1,)), ((), ())),
        preferred_element_type=jnp.float32)
    ridx = lax.broadcasted_iota(jnp.int32, (CHUNK, N), 0)
    st = jnp.where(ridx >= M - i * CHUNK, -jnp.inf, st)

    @pl.when(i == 0)
    def _bootstrap():
        ms, ams = [], []
        s = st
        for j in range(K):
            m = jnp.max(s, axis=0, keepdims=True)
            am = jnp.min(jnp.where(s == m, ridx, INT_MAX), axis=0,
                         keepdims=True)
            ms.append(m)
            ams.append(am)
            if j < K - 1:
                s = jnp.where(ridx == am, -jnp.inf, s)
        vals_ref[...] = jnp.concatenate(ms, axis=0)
        idx_ref[...] = jnp.concatenate(ams, axis=0)

    @pl.when(i > 0)
    def _threshold_extract():
        rv0 = vals_ref[...]
        ri0 = idx_ref[...]
        m0 = jnp.max(st, axis=0, keepdims=True)

        def cond(carry):
            _, rv, _, m = carry
            return jnp.any(m > rv[K - 1:K, :])

        def body(carry):
            s, rv, ri, m = carry
            am = jnp.min(jnp.where(s == m, ridx, INT_MAX), axis=0,
                         keepdims=True)
            gi = am + i * CHUNK
            ge = (rv > m) | ((rv == m) & (ri < gi))
            ge_s = jnp.concatenate(
                [jnp.ones((1, N), jnp.int32),
                 ge.astype(jnp.int32)[:K - 1, :]], axis=0) != 0
            rv_s = jnp.concatenate(
                [jnp.full((1, N), -jnp.inf, jnp.float32), rv[:K - 1, :]],
                axis=0)
            ri_s = jnp.concatenate(
                [jnp.full((1, N), INT_MAX, jnp.int32), ri[:K - 1, :]],
                axis=0)
            mb = jnp.broadcast_to(m, (K, N))
            gib = jnp.broadcast_to(gi, (K, N))
            rv = jnp.where(ge, rv, jnp.where(ge_s, mb, rv_s))
            ri = jnp.where(ge, ri, jnp.where(ge_s, gib, ri_s))
            s = jnp.where(ridx == am, -jnp.inf, s)
            m = jnp.max(s, axis=0, keepdims=True)
            return s, rv, ri, m

        _, rv, ri, _ = lax.while_loop(cond, body, (st, rv0, ri0, m0))
        vals_ref[...] = rv
        idx_ref[...] = ri


def _topk(query, queue_padded, interpret=False):
    return pl.pallas_call(
        _topk_body,
        grid=(NCHUNK,),
        in_specs=[
            pl.BlockSpec((N, D), lambda i: (0, 0)),
            pl.BlockSpec((CHUNK, D), lambda i: (i, 0)),
        ],
        out_specs=[
            pl.BlockSpec((K, N), lambda i: (0, 0)),
            pl.BlockSpec((K, N), lambda i: (0, 0)),
        ],
        out_shape=[
            jax.ShapeDtypeStruct((K, N), jnp.float32),
            jax.ShapeDtypeStruct((K, N), jnp.int32),
        ],
        compiler_params=pltpu.CompilerParams(
            dimension_semantics=("arbitrary",)),
        interpret=interpret,
    )(query, queue_padded)


_SC_WORKERS = 32
_ROWS_PER_W = (N * K) // _SC_WORKERS
_IDX_CHUNK = 128
_NJ = _ROWS_PER_W // _IDX_CHUNK


def _gather_sc(queue, flat_idx):
    mesh = plsc.VectorSubcoreMesh(core_axis_name="c", subcore_axis_name="s")

    @functools.partial(
        pl.kernel, mesh=mesh,
        out_type=jax.ShapeDtypeStruct((N * K, D), jnp.float32),
        scratch_types=[
            pltpu.VMEM((_IDX_CHUNK,), jnp.int32),
            pltpu.VMEM((_IDX_CHUNK, D), jnp.float32),
            pltpu.SemaphoreType.DMA,
        ],
        compiler_params=pltpu.CompilerParams(use_tc_tiling_on_sc=False),
    )
    def gk(table_hbm, idx_hbm, out_hbm, idx_v, rows_v, sem):
        wid = lax.axis_index("s") * 2 + lax.axis_index("c")
        base = wid * _ROWS_PER_W
        for j in range(_NJ):
            off = base + j * _IDX_CHUNK
            pltpu.sync_copy(idx_hbm.at[pl.ds(off, _IDX_CHUNK)], idx_v)
            pltpu.async_copy(table_hbm.at[idx_v], rows_v, sem).wait()
            pltpu.sync_copy(rows_v, out_hbm.at[pl.ds(off, _IDX_CHUNK)])

    return gk(queue, flat_idx)


def kernel(query, queue, k):
    queue_padded = jnp.pad(queue, ((0, MPAD - M), (0, 0)))
    vals_t, idx_t = _topk(query, queue_padded)
    values = vals_t.T
    indices = idx_t.T
    neighbors = _gather_sc(queue, indices.reshape(N * K)).reshape(N, K, D)
    values = values + (jnp.asarray(k, jnp.float32) - jnp.float32(K))
    return neighbors, values


# speedup vs baseline: 1.0581x; 1.0581x over previous
import functools

import jax
import jax.numpy as jnp
from jax import lax
from jax.experimental import pallas as pl
from jax.experimental.pallas import tpu as pltpu
from jax.experimental.pallas import tpu_sc as plsc

N = 1024
D = 32
M = 100000
K = 16
CHUNK = 512
NCHUNK = -(-M // CHUNK)
MPAD = CHUNK * NCHUNK
INT_MAX = jnp.iinfo(jnp.int32).max


def _topk_body(q_ref, t_ref, vals_ref, idx_ref):
    i = pl.program_id(0)

    st = lax.dot_general(
        t_ref[...], q_ref[...], (((1,), (1,)), ((), ())),
        preferred_element_type=jnp.float32)
    ridx = lax.broadcasted_iota(jnp.int32, (CHUNK, N), 0)
    st = jnp.where(ridx >= M - i * CHUNK, -jnp.inf, st)

    @pl.when(i == 0)
    def _bootstrap():
        ms, ams = [], []
        s = st
        for j in range(K):
            m = jnp.max(s, axis=0, keepdims=True)
            am = jnp.min(jnp.where(s == m, ridx, INT_MAX), axis=0,
                         keepdims=True)
            ms.append(m)
            ams.append(am)
            if j < K - 1:
                s = jnp.where(ridx == am, -jnp.inf, s)
        vals_ref[...] = jnp.concatenate(ms, axis=0)
        idx_ref[...] = jnp.concatenate(ams, axis=0)

    @pl.when(i > 0)
    def _threshold_extract():
        rv0 = vals_ref[...]
        ri0 = idx_ref[...]
        m0 = jnp.max(st, axis=0, keepdims=True)

        def cond(carry):
            _, rv, _, m = carry
            return jnp.any(m > rv[K - 1:K, :])

        def body(carry):
            s, rv, ri, m = carry
            am = jnp.min(jnp.where(s == m, ridx, INT_MAX), axis=0,
                         keepdims=True)
            gi = am + i * CHUNK
            ge = (rv > m) | ((rv == m) & (ri < gi))
            ge_s = jnp.concatenate(
                [jnp.ones((1, N), jnp.int32),
                 ge.astype(jnp.int32)[:K - 1, :]], axis=0) != 0
            rv_s = jnp.concatenate(
                [jnp.full((1, N), -jnp.inf, jnp.float32), rv[:K - 1, :]],
                axis=0)
            ri_s = jnp.concatenate(
                [jnp.full((1, N), INT_MAX, jnp.int32), ri[:K - 1, :]],
                axis=0)
            mb = jnp.broadcast_to(m, (K, N))
            gib = jnp.broadcast_to(gi, (K, N))
            rv = jnp.where(ge, rv, jnp.where(ge_s, mb, rv_s))
            ri = jnp.where(ge, ri, jnp.where(ge_s, gib, ri_s))
            s = jnp.where(ridx == am, -jnp.inf, s)
            m = jnp.max(s, axis=0, keepdims=True)
            return s, rv, ri, m

        _, rv, ri, _ = lax.while_loop(cond, body, (st, rv0, ri0, m0))
        vals_ref[...] = rv
        idx_ref[...] = ri


def _topk(query, queue_padded, interpret=False):
    return pl.pallas_call(
        _topk_body,
        grid=(NCHUNK,),
        in_specs=[
            pl.BlockSpec((N, D), lambda i: (0, 0)),
            pl.BlockSpec((CHUNK, D), lambda i: (i, 0)),
        ],
        out_specs=[
            pl.BlockSpec((K, N), lambda i: (0, 0)),
            pl.BlockSpec((K, N), lambda i: (0, 0)),
        ],
        out_shape=[
            jax.ShapeDtypeStruct((K, N), jnp.float32),
            jax.ShapeDtypeStruct((K, N), jnp.int32),
        ],
        compiler_params=pltpu.CompilerParams(
            dimension_semantics=("arbitrary",)),
        interpret=interpret,
    )(query, queue_padded)


_SC_WORKERS = 32
_ROWS_PER_W = (N * K) // _SC_WORKERS
_IDX_CHUNK = 128
_NJ = _ROWS_PER_W // _IDX_CHUNK


def _gather_sc(queue, flat_idx):
    mesh = plsc.VectorSubcoreMesh(core_axis_name="c", subcore_axis_name="s")

    @functools.partial(
        pl.kernel, mesh=mesh,
        out_type=jax.ShapeDtypeStruct((N * K, D), jnp.float32),
        scratch_types=[
            pltpu.VMEM((_IDX_CHUNK,), jnp.int32),
            pltpu.VMEM((_IDX_CHUNK, D), jnp.float32),
            pltpu.SemaphoreType.DMA,
        ],
        compiler_params=pltpu.CompilerParams(use_tc_tiling_on_sc=False),
    )
    def gk(table_hbm, idx_hbm, out_hbm, idx_v, rows_v, sem):
        wid = lax.axis_index("s") * 2 + lax.axis_index("c")
        base = wid * _ROWS_PER_W
        for j in range(_NJ):
            off = base + j * _IDX_CHUNK
            pltpu.sync_copy(idx_hbm.at[pl.ds(off, _IDX_CHUNK)], idx_v)
            pltpu.async_copy(table_hbm.at[idx_v], rows_v, sem).wait()
            pltpu.sync_copy(rows_v, out_hbm.at[pl.ds(off, _IDX_CHUNK)])

    return gk(queue, flat_idx)


def kernel(query, queue, k):
    queue_padded = jnp.pad(queue, ((0, MPAD - M), (0, 0)))
    vals_t, idx_t = _topk(query, queue_padded)
    values = vals_t.T
    indices = idx_t.T
    neighbors = _gather_sc(queue, indices.reshape(N * K)).reshape(N, K, D)
    values = values + (jnp.asarray(k, jnp.float32) - jnp.float32(K))
    return neighbors, values
